# baseline (speedup 1.0000x reference)
import functools

import jax
import jax.numpy as jnp
from jax import lax
from jax.experimental import pallas as pl
from jax.experimental.pallas import tpu as pltpu
from jax.experimental.pallas import tpu_sc as plsc


def _sc_fused_kernel(n_ids: int, d_model: int):
    info = plsc.get_sparse_core_info()
    nc, ns, lanes = info.num_cores, info.num_subcores, info.num_lanes
    nw = nc * ns
    per_w = n_ids // nw
    assert n_ids % nw == 0 and per_w % lanes == 0
    gchunk = 128
    assert per_w % gchunk == 0

    ch = 8
    assert per_w % (2 * ch) == 0
    npair = per_w // (2 * ch)
    celems = ch * d_model
    groups = d_model // lanes

    mesh = plsc.VectorSubcoreMesh(core_axis_name="c", subcore_axis_name="s")

    @functools.partial(
        pl.kernel,
        mesh=mesh,
        out_type=jax.ShapeDtypeStruct((n_ids * d_model,), jnp.float32),
        scratch_types=[
            pltpu.VMEM((per_w,), jnp.int32),
            pltpu.VMEM((per_w,), jnp.float32),
            pltpu.VMEM((per_w + lanes,), jnp.float32),
            pltpu.VMEM((2, celems), jnp.float32),
            pltpu.VMEM((2, celems), jnp.float32),
            pltpu.SemaphoreType.DMA,
            pltpu.SemaphoreType.DMA((2,)),
            pltpu.SemaphoreType.DMA((2,)),
        ],
    )
    def sc_fused(ids_hbm, table_hbm, hid_hbm, out_hbm,
                 idx_v, w_v, delta_v, in_t, out_t, gsem, in_sem, out_sem):
        wid = lax.axis_index("s") * nc + lax.axis_index("c")
        row0 = wid * per_w
        ebase = row0 * d_model

        pltpu.sync_copy(ids_hbm.at[pl.ds(row0, per_w)], idx_v)
        gathers = [
            pltpu.async_copy(
                table_hbm.at[idx_v.at[pl.ds(g * gchunk, gchunk)]],
                w_v.at[pl.ds(g * gchunk, gchunk)],
                gsem,
            )
            for g in range(per_w // gchunk)
        ]
        for cop in gathers:
            cop.wait()
        for i in range(per_w // lanes):
            sl = pl.ds(i * lanes, lanes)
            v = w_v[sl]
            delta_v[sl] = v - v

        def in_copy(c, slot):
            return pltpu.make_async_copy(
                hid_hbm.at[pl.ds(ebase + c * celems, celems)],
                in_t.at[slot],
                in_sem.at[slot],
            )

        def out_copy(c, slot):
            return pltpu.make_async_copy(
                out_t.at[slot],
                out_hbm.at[pl.ds(ebase + c * celems, celems)],
                out_sem.at[slot],
            )

        def compute(c, slot):
            dvec = delta_v[pl.ds(c * ch, lanes)]
            for r in range(ch):
                s = dvec[r]

                def grp_body(g, _, r=r, s=s):
                    sl = pl.ds(r * d_model + g * lanes, lanes)
                    out_t[slot, sl] = in_t[slot, sl] + s
                    return 0

                lax.fori_loop(0, groups, grp_body, 0, unroll=8)

        def pair(p, first, last):
            a, b = 2 * p, 2 * p + 1
            in_copy(b, 1).start()
            in_copy(a, 0).wait()
            if not first:
                out_copy(a - 2, 0).wait()
            compute(a, 0)
            out_copy(a, 0).start()
            if not last:
                in_copy(a + 2, 0).start()
            in_copy(b, 1).wait()
            if not first:
                out_copy(b - 2, 1).wait()
            compute(b, 1)
            out_copy(b, 1).start()

        in_copy(0, 0).start()
        if npair == 1:
            pair(0, True, True)
        else:
            pair(0, True, False)

            def loop_body(p, _):
                pair(p, False, False)
                return 0

            if npair > 2:
                lax.fori_loop(1, npair - 1, loop_body, 0)
            pair(npair - 1, False, True)
        out_copy(2 * npair - 2, 0).wait()
        out_copy(2 * npair - 1, 1).wait()

    return sc_fused


@functools.lru_cache(maxsize=None)
def _build(n_ids, d_model):
    return _sc_fused_kernel(n_ids, d_model)


def kernel(witness_ids, hidden_states, witness_weight):
    batch, seq = witness_ids.shape
    seq_h, batch_h, d_model = hidden_states.shape
    sc_fused = _build(batch * seq, d_model)
    ids_sb = witness_ids.T.reshape(-1).astype(jnp.int32)
    table = witness_weight.reshape(-1)
    out = sc_fused(ids_sb, table, hidden_states.reshape(-1))
    return out.reshape(seq_h, batch_h, d_model)


# speedup vs baseline: 1.3377x; 1.3377x over previous
import functools

import jax
import jax.numpy as jnp
from jax import lax
from jax.experimental import pallas as pl
from jax.experimental.pallas import tpu as pltpu
from jax.experimental.pallas import tpu_sc as plsc


def _sc_fused_kernel(n_ids: int, d_model: int):
    info = plsc.get_sparse_core_info()
    nc, ns, lanes = info.num_cores, info.num_subcores, info.num_lanes
    nw = nc * ns
    per_w = n_ids // nw
    assert n_ids % nw == 0 and per_w % lanes == 0
    gchunk = 128
    assert per_w % gchunk == 0

    ch = 8
    assert per_w % (2 * ch) == 0
    npair = per_w // (2 * ch)
    celems = ch * d_model
    groups = d_model // lanes

    mesh = plsc.VectorSubcoreMesh(core_axis_name="c", subcore_axis_name="s")

    @functools.partial(
        pl.kernel,
        mesh=mesh,
        out_type=jax.ShapeDtypeStruct((n_ids * d_model,), jnp.float32),
        scratch_types=[
            pltpu.VMEM((per_w,), jnp.int32),
            pltpu.VMEM((per_w,), jnp.float32),
            pltpu.VMEM((per_w + lanes,), jnp.float32),
            pltpu.VMEM((2, celems), jnp.float32),
            pltpu.VMEM((2, celems), jnp.float32),
            pltpu.SemaphoreType.DMA,
            pltpu.SemaphoreType.DMA((2,)),
            pltpu.SemaphoreType.DMA((2,)),
        ],
    )
    def sc_fused(ids_hbm, table_hbm, hid_hbm, out_hbm,
                 idx_v, w_v, delta_v, in_t, out_t, gsem, in_sem, out_sem):
        wid = lax.axis_index("s") * nc + lax.axis_index("c")
        row0 = wid * per_w
        ebase = row0 * d_model

        pltpu.sync_copy(ids_hbm.at[pl.ds(row0, per_w)], idx_v)
        gathers = [
            pltpu.async_copy(
                table_hbm.at[idx_v.at[pl.ds(g * gchunk, gchunk)]],
                w_v.at[pl.ds(g * gchunk, gchunk)],
                gsem,
            )
            for g in range(per_w // gchunk)
        ]
        for cop in gathers:
            cop.wait()
        for i in range(per_w // lanes):
            sl = pl.ds(i * lanes, lanes)
            v = w_v[sl]
            delta_v[sl] = v - v

        def in_copy(c, slot):
            return pltpu.make_async_copy(
                hid_hbm.at[pl.ds(ebase + c * celems, celems)],
                in_t.at[slot],
                in_sem.at[slot],
            )

        def out_copy(c, slot):
            return pltpu.make_async_copy(
                out_t.at[slot],
                out_hbm.at[pl.ds(ebase + c * celems, celems)],
                out_sem.at[slot],
            )

        def compute(c, slot):
            dvec = delta_v[pl.ds(c * ch, lanes)]
            for r in range(ch):
                s = dvec[r]
                base = r * d_model

                @plsc.parallel_loop(0, groups, unroll=16)
                def grp_body(g, base=base, s=s):
                    sl = pl.ds(base + g * lanes, lanes)
                    out_t[slot, sl] = in_t[slot, sl] + s

        def pair(p, first, last):
            a, b = 2 * p, 2 * p + 1
            in_copy(b, 1).start()
            in_copy(a, 0).wait()
            if not first:
                out_copy(a - 2, 0).wait()
            compute(a, 0)
            out_copy(a, 0).start()
            if not last:
                in_copy(a + 2, 0).start()
            in_copy(b, 1).wait()
            if not first:
                out_copy(b - 2, 1).wait()
            compute(b, 1)
            out_copy(b, 1).start()

        in_copy(0, 0).start()
        if npair == 1:
            pair(0, True, True)
        else:
            pair(0, True, False)

            def loop_body(p, _):
                pair(p, False, False)
                return 0

            if npair > 2:
                lax.fori_loop(1, npair - 1, loop_body, 0)
            pair(npair - 1, False, True)
        out_copy(2 * npair - 2, 0).wait()
        out_copy(2 * npair - 1, 1).wait()

    return sc_fused


@functools.lru_cache(maxsize=None)
def _build(n_ids, d_model):
    return _sc_fused_kernel(n_ids, d_model)


def kernel(witness_ids, hidden_states, witness_weight):
    batch, seq = witness_ids.shape
    seq_h, batch_h, d_model = hidden_states.shape
    sc_fused = _build(batch * seq, d_model)
    ids_sb = witness_ids.T.reshape(-1).astype(jnp.int32)
    table = witness_weight.reshape(-1)
    out = sc_fused(ids_sb, table, hidden_states.reshape(-1))
    return out.reshape(seq_h, batch_h, d_model)


# speedup vs baseline: 1.4701x; 1.0990x over previous
import functools

import jax
import jax.numpy as jnp
from jax import lax
from jax.experimental import pallas as pl
from jax.experimental.pallas import tpu as pltpu
from jax.experimental.pallas import tpu_sc as plsc


def _sc_fused_kernel(n_ids: int, d_model: int):
    info = plsc.get_sparse_core_info()
    nc, ns, lanes = info.num_cores, info.num_subcores, info.num_lanes
    nw = nc * ns
    per_w = n_ids // nw
    assert n_ids % nw == 0 and per_w % lanes == 0
    gchunk = 128
    assert per_w % gchunk == 0

    ch = 8
    assert per_w % (2 * ch) == 0
    npair = per_w // (2 * ch)
    celems = ch * d_model
    groups = d_model // lanes

    mesh = plsc.VectorSubcoreMesh(core_axis_name="c", subcore_axis_name="s")

    @functools.partial(
        pl.kernel,
        mesh=mesh,
        out_type=jax.ShapeDtypeStruct((n_ids * d_model,), jnp.float32),
        scratch_types=[
            pltpu.VMEM((per_w,), jnp.int32),
            pltpu.VMEM((per_w,), jnp.float32),
            pltpu.VMEM((per_w + lanes,), jnp.float32),
            pltpu.VMEM((2, celems), jnp.float32),
            pltpu.VMEM((2, celems), jnp.float32),
            pltpu.SemaphoreType.DMA,
            pltpu.SemaphoreType.DMA((2,)),
            pltpu.SemaphoreType.DMA((2,)),
        ],
    )
    def sc_fused(ids_hbm, table_hbm, hid_hbm, out_hbm,
                 idx_v, w_v, delta_v, in_t, out_t, gsem, in_sem, out_sem):
        wid = lax.axis_index("s") * nc + lax.axis_index("c")
        row0 = wid * per_w
        ebase = row0 * d_model

        pltpu.sync_copy(ids_hbm.at[pl.ds(row0, per_w)], idx_v)
        gathers = [
            pltpu.async_copy(
                table_hbm.at[idx_v.at[pl.ds(g * gchunk, gchunk)]],
                w_v.at[pl.ds(g * gchunk, gchunk)],
                gsem,
            )
            for g in range(per_w // gchunk)
        ]
        for cop in gathers:
            cop.wait()
        for i in range(per_w // lanes):
            sl = pl.ds(i * lanes, lanes)
            v = w_v[sl]
            delta_v[sl] = v - v

        def in_copy(c, slot):
            return pltpu.make_async_copy(
                hid_hbm.at[pl.ds(ebase + c * celems, celems)],
                in_t.at[slot],
                in_sem.at[slot],
            )

        def out_copy(c, slot):
            return pltpu.make_async_copy(
                in_t.at[slot],
                out_hbm.at[pl.ds(ebase + c * celems, celems)],
                out_sem.at[slot],
            )

        def compute(c, slot):
            pass

        def pair(p, first, last):
            a, b = 2 * p, 2 * p + 1
            in_copy(b, 1).start()
            in_copy(a, 0).wait()
            if not first:
                out_copy(a - 2, 0).wait()
            compute(a, 0)
            out_copy(a, 0).start()
            if not last:
                in_copy(a + 2, 0).start()
            in_copy(b, 1).wait()
            if not first:
                out_copy(b - 2, 1).wait()
            compute(b, 1)
            out_copy(b, 1).start()

        in_copy(0, 0).start()
        if npair == 1:
            pair(0, True, True)
        else:
            pair(0, True, False)

            def loop_body(p, _):
                pair(p, False, False)
                return 0

            if npair > 2:
                lax.fori_loop(1, npair - 1, loop_body, 0)
            pair(npair - 1, False, True)
        out_copy(2 * npair - 2, 0).wait()
        out_copy(2 * npair - 1, 1).wait()

    return sc_fused


@functools.lru_cache(maxsize=None)
def _build(n_ids, d_model):
    return _sc_fused_kernel(n_ids, d_model)


def kernel(witness_ids, hidden_states, witness_weight):
    batch, seq = witness_ids.shape
    seq_h, batch_h, d_model = hidden_states.shape
    sc_fused = _build(batch * seq, d_model)
    ids_sb = witness_ids.T.reshape(-1).astype(jnp.int32)
    table = witness_weight.reshape(-1)
    out = sc_fused(ids_sb, table, hidden_states.reshape(-1))
    return out.reshape(seq_h, batch_h, d_model)
